# Initial kernel scaffold; baseline (speedup 1.0000x reference)
import jax
import jax.numpy as jnp
from jax.experimental import pallas as pl


def kernel(vertices_0, vertices_1, vertices_2, vertices_3, processed_0, processed_1, processed_2, processed_3, coords_orig, coords_mut, W1, b1, W2, b2, Wm1, bm1, Wm2, bm2):
    raise NotImplementedError("write your pallas kernel here")


# speedup vs baseline: 2.3316x; 2.3316x over previous
import jax
import jax.numpy as jnp
from jax import lax
from jax.experimental import pallas as pl

_SIGMA2X2 = 12.5
_LOG2E = 1.4426950408889634


def _exp_accurate(x):
    y = jnp.maximum(x * _LOG2E, -126.0)
    n = jnp.floor(y + 0.5)
    f = y - n
    p = 1.5359969399e-4
    p = p * f + 1.3333557240e-3
    p = p * f + 9.6180691596e-3
    p = p * f + 5.5504108665e-2
    p = p * f + 2.4022650696e-1
    p = p * f + 6.9314718056e-1
    p = p * f + 1.0
    scale = jax.lax.bitcast_convert_type(
        (n.astype(jnp.int32) + 127) << 23, jnp.float32)
    return p * scale
_EPS = 1e-2
_VBLK = 2048
_K1 = 17
_Q = 1024


def _mspsurf_body(vt_ref, feats_ref, q_ref, qt_ref,
                  w1_ref, b1_ref, w2_ref, b2_ref,
                  wm1_ref, bm1_ref, wm2_ref, bm2_ref,
                  out_ref):
    npad = vt_ref.shape[2]
    nblk = npad // _VBLK

    projs = []
    denoms = []
    for p in range(4):
        qp = q_ref[p]
        qx = qp[:, 0:1]
        qy = qp[:, 1:2]
        qz = qp[:, 2:3]

        def body(j, carry, p=p, qx=qx, qy=qy, qz=qz):
            acc, dacc = carry
            v = vt_ref[p, :, pl.ds(j * _VBLK, _VBLK)]
            d2 = ((qx - v[0:1, :]) ** 2
                  + (qy - v[1:2, :]) ** 2
                  + (qz - v[2:3, :]) ** 2)
            w = _exp_accurate(d2 * (-1.0 / _SIGMA2X2))
            f = feats_ref[p, pl.ds(j * _VBLK, _VBLK), :]
            acc = acc + jax.lax.dot_general(
                w, f, (((1,), (0,)), ((), ())),
                preferred_element_type=jnp.float32)
            dacc = dacc + jnp.sum(w, axis=1, keepdims=True)
            return acc, dacc

        acc0 = jnp.zeros((_Q, feats_ref.shape[2]), jnp.float32)
        dacc0 = jnp.zeros((_Q, 1), jnp.float32)
        acc, dacc = lax.fori_loop(0, nblk, body, (acc0, dacc0))
        den = dacc + _EPS
        projs.append(acc / den)
        denoms.append(den)

    feats_sets = [
        jnp.concatenate([projs[0], denoms[0], projs[1], denoms[1]], axis=1),
        jnp.concatenate([projs[2], denoms[2], projs[3], denoms[3]], axis=1),
    ]

    cid = lax.broadcasted_iota(jnp.int32, (_Q, _Q), 1)
    embs = []
    for s in range(2):
        q = q_ref[2 * s]
        qt = qt_ref[s]
        q2 = jnp.sum(q * q, axis=1, keepdims=True)
        q2t = jnp.sum(qt * qt, axis=0, keepdims=True)
        g = jax.lax.dot_general(q, qt, (((1,), (0,)), ((), ())),
                                preferred_element_type=jnp.float32)
        d2 = (q2 + q2t) - 2.0 * g

        def knn_body(_, carry):
            D, A = carry
            m = jnp.min(D, axis=1, keepdims=True)
            first = jnp.min(jnp.where(D <= m, cid, jnp.int32(1 << 30)),
                            axis=1, keepdims=True)
            pick = cid == first
            A = A + pick.astype(jnp.float32)
            D = jnp.where(pick, jnp.float32(1e30), D)
            return D, A

        _, A = lax.fori_loop(0, _K1, knn_body,
                             (d2, jnp.zeros((_Q, _Q), jnp.float32)))

        feats = feats_sets[s]
        agg = jnp.dot(A, feats, preferred_element_type=jnp.float32) * (1.0 / _K1)
        h1 = jnp.maximum(
            jnp.dot(agg, w1_ref[:, :], preferred_element_type=jnp.float32)
            + b1_ref[:, :], 0.0)
        agg2 = jnp.dot(A, h1, preferred_element_type=jnp.float32) * (1.0 / _K1)
        h2 = jnp.maximum(
            jnp.dot(agg2, w2_ref[:, :], preferred_element_type=jnp.float32)
            + b2_ref[:, :], 0.0)
        embs.append(jnp.max(h2, axis=0, keepdims=True))

    x = jnp.concatenate(embs, axis=1)
    h = jnp.maximum(
        jnp.dot(x, wm1_ref[:, :], preferred_element_type=jnp.float32)
        + bm1_ref[:, :], 0.0)
    out_ref[:, :] = (jnp.dot(h, wm2_ref[:, :],
                             preferred_element_type=jnp.float32)
                     + bm2_ref[:, :])


def kernel(vertices_0, vertices_1, vertices_2, vertices_3,
           processed_0, processed_1, processed_2, processed_3,
           coords_orig, coords_mut,
           W1, b1, W2, b2, Wm1, bm1, Wm2, bm2):
    n = vertices_0.shape[0]
    npad = ((n + _VBLK - 1) // _VBLK) * _VBLK
    pad = npad - n

    verts = jnp.stack([vertices_0, vertices_1, vertices_2, vertices_3])
    verts = jnp.pad(verts, ((0, 0), (0, pad), (0, 0)), constant_values=1e6)
    vt = verts.transpose(0, 2, 1)
    feats = jnp.stack([processed_0, processed_1, processed_2, processed_3])
    feats = jnp.pad(feats, ((0, 0), (0, pad), (0, 0)))
    q = jnp.stack([coords_orig, coords_orig, coords_mut, coords_mut])
    qt = jnp.stack([coords_orig.T, coords_mut.T])

    out2d = pl.pallas_call(
        _mspsurf_body,
        out_shape=jax.ShapeDtypeStruct((1, 1), jnp.float32),
    )(vt, feats, q, qt,
      W1, b1.reshape(1, -1), W2, b2.reshape(1, -1),
      Wm1, bm1.reshape(1, -1), Wm2, bm2.reshape(1, -1))
    return out2d.reshape((1,))
